# baseline (speedup 1.0000x reference)
import functools

import jax
import jax.numpy as jnp
from jax import lax
from jax.experimental import pallas as pl
from jax.experimental.pallas import tpu as pltpu
from jax.experimental.pallas import tpu_sc as plsc

N_LEMMAS = 100000
NPH = 128
NCD = 128
B = 1024
NEG = -1e9

LB = 4096
GRID = (N_LEMMAS + LB - 1) // LB

NW = 32
BPW = B // NW
WORDS = N_LEMMAS * NCD
WPW = WORDS // NW
CHW = 25000
NCH = WPW // CHW


def _prep_body(idx_r_ref, idx_c_ref, cv_ref, pc_ref, cv2_ref, pc2_ref):
    ir = idx_r_ref[...]
    ic = idx_c_ref[...]
    same = ir == ic
    iota_c = lax.broadcasted_iota(jnp.int32, (B, B), 1)
    last = jnp.max(jnp.where(same, iota_c, -1), axis=1, keepdims=True)
    iota_r = lax.broadcasted_iota(jnp.int32, (B, 1), 0)
    is_last = last == iota_r
    onehot = (iota_c == last).astype(jnp.bfloat16)
    cv_sel = jnp.dot(onehot, cv_ref[...].astype(jnp.bfloat16),
                     preferred_element_type=jnp.float32)
    pc_sel = jnp.dot(onehot, pc_ref[...].astype(jnp.bfloat16),
                     preferred_element_type=jnp.float32)
    cv2_ref[...] = jnp.where(is_last, cv_ref[...], cv_sel)
    pc2_ref[...] = jnp.where(is_last, pc_ref[...], pc_sel)


_prep = pl.pallas_call(
    _prep_body,
    out_shape=(
        jax.ShapeDtypeStruct((B, NCD), jnp.float32),
        jax.ShapeDtypeStruct((B, NPH), jnp.float32),
    ),
)


def _sims_body(pc_ref, wt_ref, st_ref, ms_ref, wt_out_ref,
               inp_ref, ones_ref, acc_ref, semc):
    i = pl.program_id(0)
    cp = pltpu.make_async_copy(wt_ref, wt_out_ref, semc)
    cp.start()

    @pl.when(i == 0)
    def _():
        pc = pc_ref[...]
        nrm = jnp.sqrt(jnp.sum(pc * pc, axis=1, keepdims=True))
        inp_ref[...] = (pc / (nrm + 1e-8)).astype(jnp.bfloat16)
        ones_ref[...] = jnp.ones((NPH, 8), jnp.bfloat16)
        acc_ref[...] = jnp.full((B, LB), NEG, jnp.bfloat16)

    w = wt_ref[...]
    row = lax.broadcasted_iota(jnp.int32, (LB, 1), 0) + i * LB
    rvalid = row < N_LEMMAS
    wsq = (w * w).astype(jnp.bfloat16)
    nsq = jnp.dot(wsq, ones_ref[...],
                  preferred_element_type=jnp.float32)[:, 0:1]
    inv = lax.rsqrt(nsq + 1e-16)
    wn = jnp.where(rvalid, w * inv, 0.0).astype(jnp.bfloat16)
    dot = lax.dot_general(inp_ref[...], wn, (((1,), (1,)), ((), ())),
                          preferred_element_type=jnp.float32)
    lane = lax.broadcasted_iota(jnp.int32, (1, LB), 1) + i * LB
    bias = jnp.where((lane < N_LEMMAS) & (st_ref[...] > 0), 0.0, NEG)
    acc_ref[...] = jnp.maximum(acc_ref[...], (dot + bias).astype(jnp.bfloat16))

    @pl.when(i == GRID - 1)
    def _():
        ms_ref[...] = jnp.max(acc_ref[...], axis=1,
                              keepdims=True).astype(jnp.float32)

    cp.wait()


_sims = pl.pallas_call(
    _sims_body,
    grid=(GRID,),
    in_specs=[
        pl.BlockSpec((B, NPH), lambda i: (0, 0)),
        pl.BlockSpec((LB, NPH), lambda i: (i, 0)),
        pl.BlockSpec((1, LB), lambda i: (0, i)),
    ],
    out_specs=(
        pl.BlockSpec((B, 1), lambda i: (0, 0)),
        pl.BlockSpec((LB, NPH), lambda i: (i, 0)),
    ),
    out_shape=(
        jax.ShapeDtypeStruct((B, 1), jnp.float32),
        jax.ShapeDtypeStruct((N_LEMMAS, NPH), jnp.float32),
    ),
    scratch_shapes=[
        pltpu.VMEM((B, NPH), jnp.bfloat16),
        pltpu.VMEM((NPH, 8), jnp.bfloat16),
        pltpu.VMEM((B, LB), jnp.bfloat16),
        pltpu.SemaphoreType.DMA,
    ],
    compiler_params=pltpu.CompilerParams(
        dimension_semantics=("arbitrary",),
    ),
)


def _sc_copy_body(src_hbm, dst_hbm, buf0, buf1, si0, si1, so0, so1):
    wid = lax.axis_index("s") * 2 + lax.axis_index("c")
    base = wid * WPW
    bufs, sin, sout = (buf0, buf1), (si0, si1), (so0, so1)
    pending = [None, None]
    for k in range(NCH):
        b = k % 2
        if pending[b] is not None:
            pending[b].wait()
        pltpu.async_copy(src_hbm.at[pl.ds(base + k * CHW, CHW)],
                         bufs[b], sin[b]).wait()
        pending[b] = pltpu.async_copy(
            bufs[b], dst_hbm.at[pl.ds(base + k * CHW, CHW)], sout[b])
    pending[0].wait()
    pending[1].wait()


@functools.cache
def _sc_copy():
    return pl.kernel(
        _sc_copy_body,
        out_type=jax.ShapeDtypeStruct((WORDS,), jnp.float32),
        mesh=plsc.VectorSubcoreMesh(core_axis_name="c", subcore_axis_name="s",
                                    num_cores=2, num_subcores=16),
        scratch_types=[
            pltpu.VMEM((CHW,), jnp.float32),
            pltpu.VMEM((CHW,), jnp.float32),
            pltpu.SemaphoreType.DMA,
            pltpu.SemaphoreType.DMA,
            pltpu.SemaphoreType.DMA,
            pltpu.SemaphoreType.DMA,
        ],
    )


def _sc_scatter_body(idx_hbm, cv2_hbm, pc2_hbm, wcl_ref, wlpt_ref,
                     idx_v, rowc_v, rowp_v, sem):
    wid = lax.axis_index("s") * 2 + lax.axis_index("c")
    base = wid * BPW
    pltpu.sync_copy(idx_hbm.at[pl.ds(base, BPW)], idx_v)
    pltpu.sync_copy(cv2_hbm.at[pl.ds(base, BPW)], rowc_v)
    pltpu.sync_copy(pc2_hbm.at[pl.ds(base, BPW)], rowp_v)
    c1 = pltpu.async_copy(rowc_v, wcl_ref.at[idx_v], sem)
    c2 = pltpu.async_copy(rowp_v, wlpt_ref.at[idx_v], sem)
    c1.wait()
    c2.wait()


@functools.cache
def _sc_scatter():
    return pl.kernel(
        _sc_scatter_body,
        mesh=plsc.VectorSubcoreMesh(core_axis_name="c", subcore_axis_name="s",
                                    num_cores=2, num_subcores=16),
        scratch_types=[
            pltpu.VMEM((BPW,), jnp.int32),
            pltpu.VMEM((BPW, NCD), jnp.float32),
            pltpu.VMEM((BPW, NPH), jnp.float32),
            pltpu.SemaphoreType.DMA,
        ],
    )


def kernel(concept_vector, phonological_code, idx, W_C_to_L, W_L_to_P, status):
    cv2, pc2 = _prep(idx.reshape(B, 1), idx.reshape(1, B),
                     concept_vector, phonological_code)
    pc_b, wlpt_b, st_b, cv2, pc2 = lax.optimization_barrier(
        (phonological_code, W_L_to_P.T, status.reshape(1, N_LEMMAS), cv2, pc2))
    maxsim, wlpt_copy = _sims(pc_b, wlpt_b, st_b)
    wcl_copy = _sc_copy()(W_C_to_L.reshape(-1))
    wcl_ref = jax.new_ref(wcl_copy.reshape(N_LEMMAS, NCD))
    wlpt_ref = jax.new_ref(wlpt_copy)
    _sc_scatter()(idx, cv2, pc2, wcl_ref, wlpt_ref)
    return wcl_ref[...], wlpt_ref[...].T, maxsim.reshape(B)


# speedup vs baseline: 1.0318x; 1.0318x over previous
import functools

import jax
import jax.numpy as jnp
from jax import lax
from jax.experimental import pallas as pl
from jax.experimental.pallas import tpu as pltpu
from jax.experimental.pallas import tpu_sc as plsc

N_LEMMAS = 100000
NPH = 128
NCD = 128
B = 1024
NEG = -1e9

LB = 4096
GRID = (N_LEMMAS + LB - 1) // LB

NW = 32
BPW = B // NW
WORDS = N_LEMMAS * NCD
WPW = WORDS // NW
CHW = 25000
NCH = WPW // CHW


def _prep_body(idx_r_ref, idx_c_ref, cv_ref, pc_ref, cv2_ref, pc2_ref):
    ir = idx_r_ref[...]
    ic = idx_c_ref[...]
    same = ir == ic
    iota_c = lax.broadcasted_iota(jnp.int32, (B, B), 1)
    last = jnp.max(jnp.where(same, iota_c, -1), axis=1, keepdims=True)
    iota_r = lax.broadcasted_iota(jnp.int32, (B, 1), 0)
    is_last = last == iota_r
    onehot = (iota_c == last).astype(jnp.bfloat16)
    cv_sel = jnp.dot(onehot, cv_ref[...].astype(jnp.bfloat16),
                     preferred_element_type=jnp.float32)
    pc_sel = jnp.dot(onehot, pc_ref[...].astype(jnp.bfloat16),
                     preferred_element_type=jnp.float32)
    cv2_ref[...] = jnp.where(is_last, cv_ref[...], cv_sel)
    pc2_ref[...] = jnp.where(is_last, pc_ref[...], pc_sel)


_prep = pl.pallas_call(
    _prep_body,
    out_shape=(
        jax.ShapeDtypeStruct((B, NCD), jnp.float32),
        jax.ShapeDtypeStruct((B, NPH), jnp.float32),
    ),
)


def _sims_body(pc_ref, wt_ref, st_ref, ms_ref, wt_out_ref,
               inp_ref, ones_ref, acc_ref):
    i = pl.program_id(0)

    @pl.when(i == 0)
    def _():
        pc = pc_ref[...]
        nrm = jnp.sqrt(jnp.sum(pc * pc, axis=1, keepdims=True))
        inp_ref[...] = (pc / (nrm + 1e-8)).astype(jnp.bfloat16)
        ones_ref[...] = jnp.ones((NPH, 8), jnp.bfloat16)
        acc_ref[...] = jnp.full((B, LB), NEG, jnp.bfloat16)

    w = wt_ref[...]
    wt_out_ref[...] = w
    row = lax.broadcasted_iota(jnp.int32, (LB, 1), 0) + i * LB
    rvalid = row < N_LEMMAS
    wsq = (w * w).astype(jnp.bfloat16)
    nsq = jnp.dot(wsq, ones_ref[...],
                  preferred_element_type=jnp.float32)[:, 0:1]
    inv = lax.rsqrt(nsq + 1e-16)
    wn = jnp.where(rvalid, w * inv, 0.0).astype(jnp.bfloat16)
    dot = lax.dot_general(inp_ref[...], wn, (((1,), (1,)), ((), ())),
                          preferred_element_type=jnp.float32)
    lane = lax.broadcasted_iota(jnp.int32, (1, LB), 1) + i * LB
    bias = jnp.where((lane < N_LEMMAS) & (st_ref[...] > 0), 0.0, NEG)
    acc_ref[...] = jnp.maximum(acc_ref[...], (dot + bias).astype(jnp.bfloat16))

    @pl.when(i == GRID - 1)
    def _():
        ms_ref[...] = jnp.max(acc_ref[...], axis=1,
                              keepdims=True).astype(jnp.float32)


_sims = pl.pallas_call(
    _sims_body,
    grid=(GRID,),
    in_specs=[
        pl.BlockSpec((B, NPH), lambda i: (0, 0)),
        pl.BlockSpec((LB, NPH), lambda i: (i, 0)),
        pl.BlockSpec((1, LB), lambda i: (0, i)),
    ],
    out_specs=(
        pl.BlockSpec((B, 1), lambda i: (0, 0)),
        pl.BlockSpec((LB, NPH), lambda i: (i, 0)),
    ),
    out_shape=(
        jax.ShapeDtypeStruct((B, 1), jnp.float32),
        jax.ShapeDtypeStruct((N_LEMMAS, NPH), jnp.float32),
    ),
    scratch_shapes=[
        pltpu.VMEM((B, NPH), jnp.bfloat16),
        pltpu.VMEM((NPH, 8), jnp.bfloat16),
        pltpu.VMEM((B, LB), jnp.bfloat16),
    ],
    compiler_params=pltpu.CompilerParams(
        dimension_semantics=("arbitrary",),
    ),
)


def _sc_copy_body(src_hbm, dst_hbm, buf0, buf1, si0, si1, so0, so1):
    wid = lax.axis_index("s") * 2 + lax.axis_index("c")
    base = wid * WPW
    bufs, sin, sout = (buf0, buf1), (si0, si1), (so0, so1)
    pending = [None, None]
    for k in range(NCH):
        b = k % 2
        if pending[b] is not None:
            pending[b].wait()
        pltpu.async_copy(src_hbm.at[pl.ds(base + k * CHW, CHW)],
                         bufs[b], sin[b]).wait()
        pending[b] = pltpu.async_copy(
            bufs[b], dst_hbm.at[pl.ds(base + k * CHW, CHW)], sout[b])
    pending[0].wait()
    pending[1].wait()


@functools.cache
def _sc_copy():
    return pl.kernel(
        _sc_copy_body,
        out_type=jax.ShapeDtypeStruct((WORDS,), jnp.float32),
        mesh=plsc.VectorSubcoreMesh(core_axis_name="c", subcore_axis_name="s",
                                    num_cores=2, num_subcores=16),
        scratch_types=[
            pltpu.VMEM((CHW,), jnp.float32),
            pltpu.VMEM((CHW,), jnp.float32),
            pltpu.SemaphoreType.DMA,
            pltpu.SemaphoreType.DMA,
            pltpu.SemaphoreType.DMA,
            pltpu.SemaphoreType.DMA,
        ],
    )


def _sc_scatter_body(idx_hbm, cv2_hbm, pc2_hbm, wcl_ref, wlpt_ref,
                     idx_v, rowc_v, rowp_v, sem):
    wid = lax.axis_index("s") * 2 + lax.axis_index("c")
    base = wid * BPW
    pltpu.sync_copy(idx_hbm.at[pl.ds(base, BPW)], idx_v)
    pltpu.sync_copy(cv2_hbm.at[pl.ds(base, BPW)], rowc_v)
    pltpu.sync_copy(pc2_hbm.at[pl.ds(base, BPW)], rowp_v)
    c1 = pltpu.async_copy(rowc_v, wcl_ref.at[idx_v], sem)
    c2 = pltpu.async_copy(rowp_v, wlpt_ref.at[idx_v], sem)
    c1.wait()
    c2.wait()


@functools.cache
def _sc_scatter():
    return pl.kernel(
        _sc_scatter_body,
        mesh=plsc.VectorSubcoreMesh(core_axis_name="c", subcore_axis_name="s",
                                    num_cores=2, num_subcores=16),
        scratch_types=[
            pltpu.VMEM((BPW,), jnp.int32),
            pltpu.VMEM((BPW, NCD), jnp.float32),
            pltpu.VMEM((BPW, NPH), jnp.float32),
            pltpu.SemaphoreType.DMA,
        ],
    )


def kernel(concept_vector, phonological_code, idx, W_C_to_L, W_L_to_P, status):
    cv2, pc2 = _prep(idx.reshape(B, 1), idx.reshape(1, B),
                     concept_vector, phonological_code)
    maxsim, wlpt_copy = _sims(
        phonological_code, W_L_to_P.T, status.reshape(1, N_LEMMAS))
    wcl_copy = _sc_copy()(W_C_to_L.reshape(-1))
    wcl_ref = jax.new_ref(wcl_copy.reshape(N_LEMMAS, NCD))
    wlpt_ref = jax.new_ref(wlpt_copy)
    _sc_scatter()(idx, cv2, pc2, wcl_ref, wlpt_ref)
    return wcl_ref[...], wlpt_ref[...].T, maxsim.reshape(B)
